# baseline (speedup 1.0000x reference)
import jax
import jax.numpy as jnp
import numpy as np
from jax.experimental import pallas as pl
from jax.experimental.pallas import tpu as pltpu

_BINS = 10
_N = 4 * 64 * 64 * 9 * 80
_LANES = 128
_ROWS = _N // _LANES
_CB = 1024
_NCHUNKS = _ROWS // _CB
_NBUF = 4
_STRIPS = _CB // 8

_EDGES = np.arange(_BINS + 1, dtype=np.float32) / np.float32(_BINS)
_EDGES[_BINS] += np.float32(1e-6)


def _copy(hbm, buf, sem, chunk, slot, j):
    return pltpu.make_async_copy(
        hbm.at[pl.ds(chunk * _CB, _CB), :],
        buf.at[slot],
        sem.at[slot, j],
    )


def _body(lbl_hbm, x_hbm, w_hbm, out_ref, lbuf, xbuf, wbuf, acc_ref, sem):
    step = pl.program_id(0)
    slot = jax.lax.rem(step, _NBUF)

    @pl.when(step == 0)
    def _prime():
        for b in range(_NBUF - 1):
            _copy(lbl_hbm, lbuf, sem, b, b, 0).start()
            _copy(x_hbm, xbuf, sem, b, b, 1).start()
            _copy(w_hbm, wbuf, sem, b, b, 2).start()

    nxt = step + _NBUF - 1
    nslot = jax.lax.rem(nxt, _NBUF)

    @pl.when(nxt < _NCHUNKS)
    def _issue():
        _copy(lbl_hbm, lbuf, sem, nxt, nslot, 0).start()
        _copy(x_hbm, xbuf, sem, nxt, nslot, 1).start()
        _copy(w_hbm, wbuf, sem, nxt, nslot, 2).start()

    _copy(lbl_hbm, lbuf, sem, step, slot, 0).wait()
    _copy(x_hbm, xbuf, sem, step, slot, 1).wait()
    _copy(w_hbm, wbuf, sem, step, slot, 2).wait()

    def strip(base, carry):
        lbl = lbuf[slot, base:base + 8, :]
        x = xbuf[slot, base:base + 8, :]
        w = wbuf[slot, base:base + 8, :]

        s = jnp.where(lbl == 1, x, -x)
        ns = -s
        nabs = jnp.minimum(s, ns)
        e = jnp.exp(nabs)
        den = 1.0 + e
        num = jnp.where(s >= 0.0, e, 1.0)
        g = num / den
        ce = jnp.maximum(ns, 0.0) + jnp.log1p(e)

        q = jnp.where(w > 0.0, g, 2.0)

        new = [carry[0] + ce]
        for i in range(1, _BINS + 1):
            ge = q >= _EDGES[i]
            new.append(jnp.where(ge, carry[2 * i - 1] + ce, carry[2 * i - 1]))
            new.append(jnp.where(ge, carry[2 * i] + 1.0, carry[2 * i]))
        return new

    zero = jnp.zeros((8, _LANES), jnp.float32)
    accs = [zero] * (2 * _BINS + 1)
    for r in range(_STRIPS):
        accs = strip(r * 8, accs)
    stacked = jnp.stack(accs)

    @pl.when(step == 0)
    def _init():
        acc_ref[...] = stacked

    @pl.when(step != 0)
    def _accum():
        acc_ref[...] = acc_ref[...] + stacked

    @pl.when(step == _NCHUNKS - 1)
    def _fin():
        a = acc_ref[...]
        t = [jnp.sum(a[0])] + [jnp.sum(a[2 * i - 1]) for i in range(1, _BINS + 1)]
        u = [jnp.float32(_N)] + [jnp.sum(a[2 * i]) for i in range(1, _BINS + 1)]
        tot = jnp.float32(0.0)
        n = jnp.float32(0.0)
        for i in range(_BINS):
            c = u[i] - u[i + 1]
            si = t[i] - t[i + 1]
            ne = c > 0.0
            tot += jnp.where(ne, si / jnp.maximum(c, 1.0), 0.0)
            n += jnp.where(ne, 1.0, 0.0)
        out_ref[0, 0] = jnp.where(n > 0.0, tot / jnp.maximum(n, 1.0), 0.0)


def kernel(class_labels, class_logits, label_weights):
    lbl = class_labels.reshape(_ROWS, _LANES)
    x = class_logits.reshape(_ROWS, _LANES)
    w = label_weights.reshape(_ROWS, _LANES)
    out = pl.pallas_call(
        _body,
        grid=(_NCHUNKS,),
        in_specs=[
            pl.BlockSpec(memory_space=pl.ANY),
            pl.BlockSpec(memory_space=pl.ANY),
            pl.BlockSpec(memory_space=pl.ANY),
        ],
        out_specs=pl.BlockSpec(memory_space=pltpu.SMEM),
        out_shape=jax.ShapeDtypeStruct((1, 1), jnp.float32),
        scratch_shapes=[
            pltpu.VMEM((_NBUF, _CB, _LANES), jnp.int32),
            pltpu.VMEM((_NBUF, _CB, _LANES), jnp.float32),
            pltpu.VMEM((_NBUF, _CB, _LANES), jnp.float32),
            pltpu.VMEM((2 * _BINS + 1, 8, _LANES), jnp.float32),
            pltpu.SemaphoreType.DMA((_NBUF, 3)),
        ],
        compiler_params=pltpu.CompilerParams(
            dimension_semantics=("arbitrary",)),
    )(lbl, x, w)
    return out[0, 0]


# speedup vs baseline: 2.3384x; 2.3384x over previous
import jax
import jax.numpy as jnp
import numpy as np
from jax.experimental import pallas as pl
from jax.experimental.pallas import tpu as pltpu

_BINS = 10
_R = 16384
_C = 720
_CPAD = 768
_NPAD = _R * _CPAD
_BR = 2048
_GRID = _R // _BR
_STRIPS = _BR // 8

_EDGES = np.arange(_BINS + 1, dtype=np.float32) / np.float32(_BINS)
_EDGES[_BINS] += np.float32(1e-6)


def _body(lbl_ref, x_ref, w_ref, out_ref, acc_ref):
    step = pl.program_id(0)

    zf32 = jnp.zeros((8, _CPAD - _C), jnp.float32)
    zi32 = jnp.zeros((8, _CPAD - _C), jnp.int32)

    def substrip(lbl, x, w, carry):
        s = jnp.where(lbl == 1, x, -x)
        ns = -s
        nabs = jnp.minimum(s, ns)
        e = jnp.exp(nabs)
        den = 1.0 + e
        num = jnp.where(s >= 0.0, e, 1.0)
        g = num / den
        ce = jnp.maximum(ns, 0.0) + jnp.log1p(e)

        q = jnp.where(w > 0.0, g, 2.0)

        new = [carry[0] + ce]
        for i in range(1, _BINS + 1):
            ge = q >= _EDGES[i]
            new.append(jnp.where(ge, carry[2 * i - 1] + ce, carry[2 * i - 1]))
            new.append(jnp.where(ge, carry[2 * i] + 1.0, carry[2 * i]))
        return new

    def strip(base, carry):
        for k in range(5):
            lo = k * 128
            carry = substrip(
                lbl_ref[base:base + 8, lo:lo + 128],
                x_ref[base:base + 8, lo:lo + 128],
                w_ref[base:base + 8, lo:lo + 128],
                carry,
            )
        lbl = jnp.concatenate([lbl_ref[base:base + 8, 640:_C], zi32], axis=1)
        x = jnp.concatenate([x_ref[base:base + 8, 640:_C], zf32], axis=1)
        w = jnp.concatenate([w_ref[base:base + 8, 640:_C], zf32], axis=1)
        return substrip(lbl, x, w, carry)

    zero = jnp.zeros((8, 128), jnp.float32)
    accs = [zero] * (2 * _BINS + 1)
    for r in range(_STRIPS):
        accs = strip(r * 8, accs)
    stacked = jnp.stack(accs)

    @pl.when(step == 0)
    def _init():
        acc_ref[...] = stacked

    @pl.when(step != 0)
    def _accum():
        acc_ref[...] = acc_ref[...] + stacked

    @pl.when(step == _GRID - 1)
    def _fin():
        a = acc_ref[...]
        t = [jnp.sum(a[0])] + [jnp.sum(a[2 * i - 1]) for i in range(1, _BINS + 1)]
        u = [jnp.float32(_NPAD)] + [jnp.sum(a[2 * i]) for i in range(1, _BINS + 1)]
        tot = jnp.float32(0.0)
        n = jnp.float32(0.0)
        for i in range(_BINS):
            c = u[i] - u[i + 1]
            si = t[i] - t[i + 1]
            ne = c > 0.0
            tot += jnp.where(ne, si / jnp.maximum(c, 1.0), 0.0)
            n += jnp.where(ne, 1.0, 0.0)
        out_ref[0, 0] = jnp.where(n > 0.0, tot / jnp.maximum(n, 1.0), 0.0)


def kernel(class_labels, class_logits, label_weights):
    lbl = class_labels.reshape(_R, _C)
    x = class_logits.reshape(_R, _C)
    w = label_weights.reshape(_R, _C)
    out = pl.pallas_call(
        _body,
        grid=(_GRID,),
        in_specs=[
            pl.BlockSpec((_BR, _C), lambda i: (i, 0)),
            pl.BlockSpec((_BR, _C), lambda i: (i, 0)),
            pl.BlockSpec((_BR, _C), lambda i: (i, 0)),
        ],
        out_specs=pl.BlockSpec(memory_space=pltpu.SMEM),
        out_shape=jax.ShapeDtypeStruct((1, 1), jnp.float32),
        scratch_shapes=[pltpu.VMEM((2 * _BINS + 1, 8, 128), jnp.float32)],
        compiler_params=pltpu.CompilerParams(
            dimension_semantics=("arbitrary",)),
    )(lbl, x, w)
    return out[0, 0]


# speedup vs baseline: 2.3433x; 1.0021x over previous
import jax
import jax.numpy as jnp
import numpy as np
from jax.experimental import pallas as pl
from jax.experimental.pallas import tpu as pltpu

_BINS = 10
_R = 16384
_C = 720
_CPAD = 768
_NPAD = _R * _CPAD
_BR = 2048
_GRID = _R // _BR
_STRIPS = _BR // 8

_EDGES = np.arange(_BINS + 1, dtype=np.float32) / np.float32(_BINS)
_EDGES[_BINS] += np.float32(1e-6)

_L = [None] + [np.float32(np.log((1.0 - np.float64(_EDGES[i])) / np.float64(_EDGES[i])))
               for i in range(1, _BINS)] + [np.float32(-np.inf)]
_NEGBIG = np.float32(-np.inf)


def _body(lbl_ref, x_ref, w_ref, out_ref, acc_ref):
    step = pl.program_id(0)

    zf32 = jnp.zeros((8, _CPAD - _C), jnp.float32)
    zi32 = jnp.zeros((8, _CPAD - _C), jnp.int32)

    def substrip(lbl, x, w, carry):
        s = jnp.where(lbl == 1, x, -x)
        ns = -s
        nabs = jnp.minimum(s, ns)
        e = jnp.exp(nabs)
        ce = jnp.maximum(ns, 0.0) + jnp.log1p(e)

        sq = jnp.where(w > 0.0, s, _NEGBIG)

        new = [carry[0] + ce]
        for i in range(1, _BINS + 1):
            ge = sq <= _L[i]
            new.append(jnp.where(ge, carry[2 * i - 1] + ce, carry[2 * i - 1]))
            new.append(jnp.where(ge, carry[2 * i] + 1.0, carry[2 * i]))
        return new

    def strip(base, carry):
        for k in range(5):
            lo = k * 128
            carry = substrip(
                lbl_ref[base:base + 8, lo:lo + 128],
                x_ref[base:base + 8, lo:lo + 128],
                w_ref[base:base + 8, lo:lo + 128],
                carry,
            )
        lbl = jnp.concatenate([lbl_ref[base:base + 8, 640:_C], zi32], axis=1)
        x = jnp.concatenate([x_ref[base:base + 8, 640:_C], zf32], axis=1)
        w = jnp.concatenate([w_ref[base:base + 8, 640:_C], zf32], axis=1)
        return substrip(lbl, x, w, carry)

    zero = jnp.zeros((8, 128), jnp.float32)
    accs = [zero] * (2 * _BINS + 1)
    for r in range(_STRIPS):
        accs = strip(r * 8, accs)
    stacked = jnp.stack(accs)

    @pl.when(step == 0)
    def _init():
        acc_ref[...] = stacked

    @pl.when(step != 0)
    def _accum():
        acc_ref[...] = acc_ref[...] + stacked

    @pl.when(step == _GRID - 1)
    def _fin():
        a = acc_ref[...]
        t = [jnp.sum(a[0])] + [jnp.sum(a[2 * i - 1]) for i in range(1, _BINS + 1)]
        u = [jnp.float32(_NPAD)] + [jnp.sum(a[2 * i]) for i in range(1, _BINS + 1)]
        tot = jnp.float32(0.0)
        n = jnp.float32(0.0)
        for i in range(_BINS):
            c = u[i] - u[i + 1]
            si = t[i] - t[i + 1]
            ne = c > 0.0
            tot += jnp.where(ne, si / jnp.maximum(c, 1.0), 0.0)
            n += jnp.where(ne, 1.0, 0.0)
        out_ref[0, 0] = jnp.where(n > 0.0, tot / jnp.maximum(n, 1.0), 0.0)


def kernel(class_labels, class_logits, label_weights):
    lbl = class_labels.reshape(_R, _C)
    x = class_logits.reshape(_R, _C)
    w = label_weights.reshape(_R, _C)
    out = pl.pallas_call(
        _body,
        grid=(_GRID,),
        in_specs=[
            pl.BlockSpec((_BR, _C), lambda i: (i, 0)),
            pl.BlockSpec((_BR, _C), lambda i: (i, 0)),
            pl.BlockSpec((_BR, _C), lambda i: (i, 0)),
        ],
        out_specs=pl.BlockSpec(memory_space=pltpu.SMEM),
        out_shape=jax.ShapeDtypeStruct((1, 1), jnp.float32),
        scratch_shapes=[pltpu.VMEM((2 * _BINS + 1, 8, 128), jnp.float32)],
        compiler_params=pltpu.CompilerParams(
            dimension_semantics=("arbitrary",)),
    )(lbl, x, w)
    return out[0, 0]


# speedup vs baseline: 2.3930x; 1.0212x over previous
import jax
import jax.numpy as jnp
import numpy as np
from jax.experimental import pallas as pl
from jax.experimental.pallas import tpu as pltpu

_BINS = 10
_R = 16384
_C = 720
_CPAD = 768
_NPAD = _R * _CPAD
_BR = 1024
_GRID = _R // _BR
_STRIPS = _BR // 8

_EDGES = np.arange(_BINS + 1, dtype=np.float32) / np.float32(_BINS)
_EDGES[_BINS] += np.float32(1e-6)

_L = [None] + [np.float32(np.log((1.0 - np.float64(_EDGES[i])) / np.float64(_EDGES[i])))
               for i in range(1, _BINS)] + [np.float32(-np.inf)]
_NEGBIG = np.float32(-np.inf)


def _body(lbl_ref, x_ref, w_ref, out_ref, acc_ref):
    step = pl.program_id(0)

    zf32 = jnp.zeros((8, _CPAD - _C), jnp.float32)
    zi32 = jnp.zeros((8, _CPAD - _C), jnp.int32)

    def substrip(lbl, x, w, carry):
        s = jnp.where(lbl == 1, x, -x)
        ns = -s
        nabs = jnp.minimum(s, ns)
        e = jnp.exp(nabs)
        ce = jnp.maximum(ns, 0.0) + jnp.log1p(e)

        sq = jnp.where(w > 0.0, s, _NEGBIG)

        new = [carry[0] + ce]
        for i in range(1, _BINS + 1):
            ge = sq <= _L[i]
            new.append(jnp.where(ge, carry[2 * i - 1] + ce, carry[2 * i - 1]))
            new.append(jnp.where(ge, carry[2 * i] + 1.0, carry[2 * i]))
        return new

    def strip(base, carry):
        for k in range(5):
            lo = k * 128
            carry = substrip(
                lbl_ref[base:base + 8, lo:lo + 128],
                x_ref[base:base + 8, lo:lo + 128],
                w_ref[base:base + 8, lo:lo + 128],
                carry,
            )
        lbl = jnp.concatenate([lbl_ref[base:base + 8, 640:_C], zi32], axis=1)
        x = jnp.concatenate([x_ref[base:base + 8, 640:_C], zf32], axis=1)
        w = jnp.concatenate([w_ref[base:base + 8, 640:_C], zf32], axis=1)
        return substrip(lbl, x, w, carry)

    zero = jnp.zeros((8, 128), jnp.float32)
    accs = [zero] * (2 * _BINS + 1)
    for r in range(_STRIPS):
        accs = strip(r * 8, accs)
    stacked = jnp.stack(accs)

    @pl.when(step == 0)
    def _init():
        acc_ref[...] = stacked

    @pl.when(step != 0)
    def _accum():
        acc_ref[...] = acc_ref[...] + stacked

    @pl.when(step == _GRID - 1)
    def _fin():
        a = acc_ref[...]
        t = [jnp.sum(a[0])] + [jnp.sum(a[2 * i - 1]) for i in range(1, _BINS + 1)]
        u = [jnp.float32(_NPAD)] + [jnp.sum(a[2 * i]) for i in range(1, _BINS + 1)]
        tot = jnp.float32(0.0)
        n = jnp.float32(0.0)
        for i in range(_BINS):
            c = u[i] - u[i + 1]
            si = t[i] - t[i + 1]
            ne = c > 0.0
            tot += jnp.where(ne, si / jnp.maximum(c, 1.0), 0.0)
            n += jnp.where(ne, 1.0, 0.0)
        out_ref[0, 0] = jnp.where(n > 0.0, tot / jnp.maximum(n, 1.0), 0.0)


def kernel(class_labels, class_logits, label_weights):
    lbl = class_labels.reshape(_R, _C)
    x = class_logits.reshape(_R, _C)
    w = label_weights.reshape(_R, _C)
    out = pl.pallas_call(
        _body,
        grid=(_GRID,),
        in_specs=[
            pl.BlockSpec((_BR, _C), lambda i: (i, 0)),
            pl.BlockSpec((_BR, _C), lambda i: (i, 0)),
            pl.BlockSpec((_BR, _C), lambda i: (i, 0)),
        ],
        out_specs=pl.BlockSpec(memory_space=pltpu.SMEM),
        out_shape=jax.ShapeDtypeStruct((1, 1), jnp.float32),
        scratch_shapes=[pltpu.VMEM((2 * _BINS + 1, 8, 128), jnp.float32)],
        compiler_params=pltpu.CompilerParams(
            dimension_semantics=("arbitrary",)),
    )(lbl, x, w)
    return out[0, 0]
